# baseline (speedup 1.0000x reference)
import functools

import jax
import jax.numpy as jnp
from jax import lax
from jax.experimental import pallas as pl
from jax.experimental.pallas import tpu as pltpu
from jax.experimental.pallas import tpu_sc as plsc

NUM_EDGES = 1000000
MEMORY_DIM = 64
BATCH = 16384

_info = plsc.get_sparse_core_info()
_NC, _NS = _info.num_cores, _info.num_subcores
_NW = _NC * _NS
_B_PER_W = BATCH // _NW
_CHUNK = 128
_NCHUNKS = _B_PER_W // _CHUNK


def _body(idx_hbm, mem_hbm, lu_hbm, mem_out_hbm, lu_out_hbm,
          idx_v, rows_v, lu_v, sem):
    wid = lax.axis_index("s") * _NC + lax.axis_index("c")
    base = wid * _B_PER_W
    pltpu.sync_copy(idx_hbm.at[wid], idx_v)
    copies = []
    for j in range(_NCHUNKS):
        dst = rows_v.at[pl.ds(j * _CHUNK, _CHUNK)]
        copies.append(pltpu.async_copy(mem_hbm.at[idx_v.at[j]], dst, sem))
    for j in range(_NCHUNKS):
        dst = lu_v.at[pl.ds(j * _CHUNK, _CHUNK)]
        copies.append(pltpu.async_copy(lu_hbm.at[idx_v.at[j]], dst, sem))
    for c in copies:
        c.wait()
    pltpu.sync_copy(rows_v, mem_out_hbm.at[pl.ds(base, _B_PER_W)])
    pltpu.sync_copy(lu_v, lu_out_hbm.at[pl.ds(base, _B_PER_W)])


@jax.jit
def _gather(idx3, memory, last_update):
    mesh = plsc.VectorSubcoreMesh(core_axis_name="c", subcore_axis_name="s")
    return pl.kernel(
        _body,
        mesh=mesh,
        out_type=(
            jax.ShapeDtypeStruct((BATCH, MEMORY_DIM), jnp.float32),
            jax.ShapeDtypeStruct((BATCH,), jnp.int32),
        ),
        scratch_types=[
            pltpu.VMEM((_NCHUNKS, _CHUNK), jnp.int32),
            pltpu.VMEM((_B_PER_W, MEMORY_DIM), jnp.float32),
            pltpu.VMEM((_B_PER_W,), jnp.int32),
            pltpu.SemaphoreType.DMA,
        ],
        compiler_params=pltpu.CompilerParams(use_tc_tiling_on_sc=False),
    )(idx3, memory, last_update)


def kernel(e_id, memory, last_update):
    idx3 = e_id.astype(jnp.int32).reshape(_NW, _NCHUNKS, _CHUNK)
    mem_out, lu_out = _gather(idx3, memory, last_update)
    return (mem_out, lu_out.astype(last_update.dtype))
